# baseline (speedup 1.0000x reference)
import functools

import jax
import jax.numpy as jnp
import numpy as np
from jax import lax
from jax.experimental import pallas as pl
from jax.experimental.pallas import tpu as pltpu
from jax.experimental.pallas import tpu_sc as plsc

_VOCAB = 100000
_D = 64
_SEQ = 200
_BATCH = 4096
_SCALE = float(np.sqrt(_D))

_NC = 2
_NS = 16
_NW = _NC * _NS

_BB = _BATCH // _NW
_NRING = 4


def _pos_encoding() -> jnp.ndarray:
    position = np.arange(_SEQ)[:, np.newaxis]
    div_term = np.exp(np.arange(0, _D, 2) * -(np.log(10000.0) / _D))
    pe = np.zeros((_SEQ, _D), dtype=np.float32)
    pe[:, 0::2] = np.sin(position * div_term)
    pe[:, 1::2] = np.cos(position * div_term)
    return jnp.asarray(pe)


def _sc_kernel(x_hbm, pe_hbm, w_hbm, out_hbm, xblk_v, pe_v, gbufs, obufs,
               gsems, ssems):
    wid = lax.axis_index("s") * _NC + lax.axis_index("c")

    pltpu.sync_copy(x_hbm.at[:, wid], xblk_v)
    pltpu.sync_copy(pe_hbm, pe_v)

    lanes = lax.iota(jnp.int32, 16)

    def fire_gather(s, slot):
        pltpu.async_copy(
            w_hbm.at[xblk_v.at[s // 8, s % 8]], gbufs[slot], gsems[slot])

    def drain_gather(slot):
        pltpu.make_async_copy(
            w_hbm.at[pl.ds(0, _BB)], gbufs[slot], gsems[slot]).wait()

    def fire_store(s, slot):
        pltpu.async_copy(obufs[slot], out_hbm.at[s, :, wid], ssems[slot])

    def drain_store(slot):
        pltpu.make_async_copy(
            obufs[slot], out_hbm.at[0, :, 0], ssems[slot]).wait()

    def compute(s, slot):
        g = gbufs[slot]
        o = obufs[slot]
        for t in range(_D // 16):
            pe16 = pe_v[s, pl.ds(t * 16, 16)]

            def dd_body(dd, _, t=t, pe16=pe16):
                d = t * 16 + dd
                splat = lax.gather(
                    pe16, jnp.full((16, 1), dd, jnp.int32),
                    dimension_numbers=lax.GatherDimensionNumbers(
                        offset_dims=(), collapsed_slice_dims=(0,),
                        start_index_map=(0,)),
                    slice_sizes=(1,),
                    mode=lax.GatherScatterMode.PROMISE_IN_BOUNDS)
                dcol = jnp.full((16,), d, jnp.int32)
                for j in range(_BB // 16):
                    v = plsc.load_gather(g, [j * 16 + lanes, dcol])
                    o[d // 8, d % 8, pl.ds(j * 16, 16)] = v * _SCALE + splat
                return _

            lax.fori_loop(0, 16, dd_body, 0)

    for q in range(_NRING - 1):
        fire_gather(q, q)

    def outer_body(i, _):
        for b in range(_NRING):
            q = i * _NRING + b
            drain_gather(b)
            nslot = (b + _NRING - 1) % _NRING

            @pl.when(q + _NRING - 1 < _SEQ)
            def _refill():
                @pl.when(q >= 1)
                def _wait_prev_store():
                    drain_store(nslot)
                fire_gather(q + _NRING - 1, nslot)

            compute(q, b)
            fire_store(q, b)
        return _

    lax.fori_loop(0, _SEQ // _NRING, outer_body, 0)

    for b in range(_NRING):
        drain_store(b)


@jax.jit
def _run(x_phys, pe, W):
    mesh = plsc.VectorSubcoreMesh(core_axis_name="c", subcore_axis_name="s")
    f = functools.partial(
        pl.kernel,
        mesh=mesh,
        out_type=jax.ShapeDtypeStruct((_SEQ, _D // 8, _NW, 8, _BB),
                                      jnp.float32),
        scratch_types=[
            pltpu.VMEM((_SEQ // 8, 8, _BB), jnp.int32),
            pltpu.VMEM((_SEQ, _D), jnp.float32),
            [pltpu.VMEM((_BB, _D), jnp.float32)] * _NRING,
            [pltpu.VMEM((_D // 8, 8, _BB), jnp.float32)] * _NRING,
            [pltpu.SemaphoreType.DMA] * _NRING,
            [pltpu.SemaphoreType.DMA] * _NRING,
        ],
        compiler_params=pltpu.CompilerParams(
            use_tc_tiling_on_sc=False, needs_layout_passes=False),
    )(_sc_kernel)
    return f(x_phys, pe, W)


def kernel(x, W):
    x_phys = x.reshape(_NW, _BB, _SEQ // 8, 8).transpose(2, 0, 3, 1)
    phys = _run(x_phys, _pos_encoding(), W)
    return phys.transpose(2, 4, 0, 1, 3).reshape(_BATCH, _SEQ, _D)


# speedup vs baseline: 1.4618x; 1.4618x over previous
import functools

import jax
import jax.numpy as jnp
import numpy as np
from jax import lax
from jax.experimental import pallas as pl
from jax.experimental.pallas import tpu as pltpu
from jax.experimental.pallas import tpu_sc as plsc

_VOCAB = 100000
_D = 64
_SEQ = 200
_BATCH = 4096
_SCALE = float(np.sqrt(_D))

_NC = 2
_NS = 16
_NW = _NC * _NS

_BB = _BATCH // _NW
_NRING = 5


def _pos_encoding() -> jnp.ndarray:
    position = np.arange(_SEQ)[:, np.newaxis]
    div_term = np.exp(np.arange(0, _D, 2) * -(np.log(10000.0) / _D))
    pe = np.zeros((_SEQ, _D), dtype=np.float32)
    pe[:, 0::2] = np.sin(position * div_term)
    pe[:, 1::2] = np.cos(position * div_term)
    return jnp.asarray(pe)


def _sc_kernel(x_hbm, pe_hbm, w_hbm, out_hbm, xblk_v, pe_v, gbufs, obufs,
               gsems, ssems):
    wid = lax.axis_index("s") * _NC + lax.axis_index("c")

    pltpu.sync_copy(x_hbm.at[:, wid], xblk_v)
    pltpu.sync_copy(pe_hbm, pe_v)

    lanes = lax.iota(jnp.int32, 16)

    def fire_gather(s, slot):
        pltpu.async_copy(
            w_hbm.at[xblk_v.at[s // 8, s % 8]], gbufs[slot], gsems[slot])

    def drain_gather(slot):
        pltpu.make_async_copy(
            w_hbm.at[pl.ds(0, _BB)], gbufs[slot], gsems[slot]).wait()

    def fire_store(s, slot):
        pltpu.async_copy(obufs[slot], out_hbm.at[s, :, wid], ssems[slot])

    def drain_store(slot):
        pltpu.make_async_copy(
            obufs[slot], out_hbm.at[0, :, 0], ssems[slot]).wait()

    def compute(s, slot):
        g = gbufs[slot]
        o = obufs[slot]
        for t in range(_D // 16):
            pe16 = pe_v[s, pl.ds(t * 16, 16)]

            def dd_body(dd, _, t=t, pe16=pe16):
                d = t * 16 + dd
                splat = lax.gather(
                    pe16, jnp.full((16, 1), dd, jnp.int32),
                    dimension_numbers=lax.GatherDimensionNumbers(
                        offset_dims=(), collapsed_slice_dims=(0,),
                        start_index_map=(0,)),
                    slice_sizes=(1,),
                    mode=lax.GatherScatterMode.PROMISE_IN_BOUNDS)
                dcol = jnp.full((16,), d, jnp.int32)
                vals = [plsc.load_gather(g, [j * 16 + lanes, dcol])
                        for j in range(_BB // 16)]
                for j in range(_BB // 16):
                    o[d // 8, d % 8, pl.ds(j * 16, 16)] = (
                        vals[j] * _SCALE + splat)
                return _

            lax.fori_loop(0, 16, dd_body, 0)

    for q in range(_NRING - 1):
        fire_gather(q, q)

    def outer_body(i, _):
        for b in range(_NRING):
            q = i * _NRING + b
            drain_gather(b)
            compute(q, b)
            fire_store(q, b)
            nslot = (b + _NRING - 1) % _NRING

            @pl.when(q + _NRING - 1 < _SEQ)
            def _refill():
                @pl.when(q >= 1)
                def _wait_prev_store():
                    drain_store(nslot)
                fire_gather(q + _NRING - 1, nslot)
        return _

    lax.fori_loop(0, _SEQ // _NRING, outer_body, 0)

    for b in range(_NRING):
        drain_store(b)


@jax.jit
def _run(x_phys, pe, W):
    mesh = plsc.VectorSubcoreMesh(core_axis_name="c", subcore_axis_name="s")
    f = functools.partial(
        pl.kernel,
        mesh=mesh,
        out_type=jax.ShapeDtypeStruct((_SEQ, _D // 8, _NW, 8, _BB),
                                      jnp.float32),
        scratch_types=[
            pltpu.VMEM((_SEQ // 8, 8, _BB), jnp.int32),
            pltpu.VMEM((_SEQ, _D), jnp.float32),
            [pltpu.VMEM((_BB, _D), jnp.float32)] * _NRING,
            [pltpu.VMEM((_D // 8, 8, _BB), jnp.float32)] * _NRING,
            [pltpu.SemaphoreType.DMA] * _NRING,
            [pltpu.SemaphoreType.DMA] * _NRING,
        ],
        compiler_params=pltpu.CompilerParams(
            use_tc_tiling_on_sc=False, needs_layout_passes=False),
    )(_sc_kernel)
    return f(x_phys, pe, W)


def kernel(x, W):
    x_phys = x.reshape(_NW, _BB, _SEQ // 8, 8).transpose(2, 0, 3, 1)
    phys = _run(x_phys, _pos_encoding(), W)
    return phys.transpose(2, 4, 0, 1, 3).reshape(_BATCH, _SEQ, _D)


# speedup vs baseline: 2.2949x; 1.5699x over previous
import functools

import jax
import jax.numpy as jnp
import numpy as np
from jax import lax
from jax.experimental import pallas as pl
from jax.experimental.pallas import tpu as pltpu
from jax.experimental.pallas import tpu_sc as plsc

_VOCAB = 100000
_D = 64
_SEQ = 200
_BATCH = 4096
_SCALE = float(np.sqrt(_D))

_NC = 2
_NS = 16
_NW = _NC * _NS

_BB = _BATCH // _NW
_NRING = 5
_BP = _BB + 1


def _pos_encoding() -> jnp.ndarray:
    position = np.arange(_SEQ)[:, np.newaxis]
    div_term = np.exp(np.arange(0, _D, 2) * -(np.log(10000.0) / _D))
    pe = np.zeros((_SEQ, _D), dtype=np.float32)
    pe[:, 0::2] = np.sin(position * div_term)
    pe[:, 1::2] = np.cos(position * div_term)
    return jnp.asarray(pe)


def _sc_kernel(x_hbm, pe_hbm, w_hbm, out_hbm, xblk_v, pe_v, gbufs, obufs,
               gsems, ssems):
    wid = lax.axis_index("s") * _NC + lax.axis_index("c")

    pltpu.sync_copy(x_hbm.at[:, wid], xblk_v)
    pltpu.sync_copy(pe_hbm, pe_v)

    lanes = lax.iota(jnp.int32, 16)

    def fire_gather(s, slot):
        pltpu.async_copy(
            w_hbm.at[xblk_v.at[s // 8, s % 8]], gbufs[slot], gsems[slot])

    def drain_gather(slot):
        pltpu.make_async_copy(
            w_hbm.at[pl.ds(0, _BB)], gbufs[slot], gsems[slot]).wait()

    def fire_store(s, slot):
        pltpu.async_copy(
            obufs[slot].at[:, :, pl.ds(0, _BB)],
            out_hbm.at[s, :, wid], ssems[slot])

    def drain_store(slot):
        pltpu.make_async_copy(
            obufs[slot].at[:, :, pl.ds(0, _BB)],
            out_hbm.at[0, :, 0], ssems[slot]).wait()

    def compute(s, slot):
        g = gbufs[slot]
        o = obufs[slot]
        pes = [pe_v[s, pl.ds(t * 16, 16)] for t in range(_D // 16)]
        d0 = [(t * 16 + lanes) // 8 for t in range(_D // 16)]
        d1 = [(t * 16 + lanes) % 8 for t in range(_D // 16)]

        def row_body(r, _):
            rc = jnp.full((16,), r, jnp.int32)
            for t in range(_D // 16):
                v = g[r, pl.ds(t * 16, 16)]
                plsc.store_scatter(o, [d0[t], d1[t], rc],
                                   v * _SCALE + pes[t])
            return _

        lax.fori_loop(0, _BB, row_body, 0)

    for q in range(_NRING - 1):
        fire_gather(q, q)

    def outer_body(i, _):
        for b in range(_NRING):
            q = i * _NRING + b
            drain_gather(b)
            compute(q, b)
            fire_store(q, b)
            nslot = (b + _NRING - 1) % _NRING

            @pl.when(q + _NRING - 1 < _SEQ)
            def _refill():
                @pl.when(q >= 1)
                def _wait_prev_store():
                    drain_store(nslot)
                fire_gather(q + _NRING - 1, nslot)
        return _

    lax.fori_loop(0, _SEQ // _NRING, outer_body, 0)

    for b in range(_NRING):
        drain_store(b)


@jax.jit
def _run(x_phys, pe, W):
    mesh = plsc.VectorSubcoreMesh(core_axis_name="c", subcore_axis_name="s")
    f = functools.partial(
        pl.kernel,
        mesh=mesh,
        out_type=jax.ShapeDtypeStruct((_SEQ, _D // 8, _NW, 8, _BB),
                                      jnp.float32),
        scratch_types=[
            pltpu.VMEM((_SEQ // 8, 8, _BB), jnp.int32),
            pltpu.VMEM((_SEQ, _D), jnp.float32),
            [pltpu.VMEM((_BB, _D), jnp.float32)] * _NRING,
            [pltpu.VMEM((_D // 8, 8, _BP), jnp.float32)] * _NRING,
            [pltpu.SemaphoreType.DMA] * _NRING,
            [pltpu.SemaphoreType.DMA] * _NRING,
        ],
        compiler_params=pltpu.CompilerParams(
            use_tc_tiling_on_sc=False, needs_layout_passes=False),
    )(_sc_kernel)
    return f(x_phys, pe, W)


def kernel(x, W):
    x_phys = x.reshape(_NW, _BB, _SEQ // 8, 8).transpose(2, 0, 3, 1)
    phys = _run(x_phys, _pos_encoding(), W)
    return phys.transpose(2, 4, 0, 1, 3).reshape(_BATCH, _SEQ, _D)
